# Initial kernel scaffold; baseline (speedup 1.0000x reference)
import jax
import jax.numpy as jnp
from jax.experimental import pallas as pl


def kernel(X, batch, Q, ln_w, ln_b, W1, b1, W2, b2):
    raise NotImplementedError("write your pallas kernel here")


# speedup vs baseline: 1.2913x; 1.2913x over previous
import functools

import jax
import jax.numpy as jnp
from jax import lax
from jax.experimental import pallas as pl
from jax.experimental.pallas import tpu as pltpu
from jax.experimental.pallas import tpu_sc as plsc

_B = 1000
_BPAD = 1024
_TA = 1024
_TS = 512
_NW = 32
_ACC = _BPAD * 32


def _decompose_mlp_body(x_ref, lnw_ref, lnb_ref, w1_ref, b1_ref, w2_ref,
                        b2_ref, cf_ref):
    xs = [x_ref[:, e, :] for e in range(9)]
    m = (xs[0] + xs[4] + xs[8]) * (1.0 / 3.0)
    d1 = xs[1] - xs[3]
    d2 = xs[2] - xs[6]
    d5 = xs[5] - xs[7]
    na = 0.5 * (d1 * d1 + d2 * d2 + d5 * d5)
    s1 = xs[1] + xs[3]
    s2 = xs[2] + xs[6]
    s5 = xs[5] + xs[7]
    t0 = xs[0] - m
    t4 = xs[4] - m
    t8 = xs[8] - m
    ns = t0 * t0 + t4 * t4 + t8 * t8 + 0.5 * (s1 * s1 + s2 * s2 + s5 * s5)
    ssum = (jnp.sum(m, axis=1, keepdims=True)
            + jnp.sum(na, axis=1, keepdims=True)
            + jnp.sum(ns, axis=1, keepdims=True))
    mu = ssum * (1.0 / 384.0)
    dm = m - mu
    dna = na - mu
    dns = ns - mu
    var = (jnp.sum(dm * dm, axis=1, keepdims=True)
           + jnp.sum(dna * dna, axis=1, keepdims=True)
           + jnp.sum(dns * dns, axis=1, keepdims=True)) * (1.0 / 384.0)
    inv = lax.rsqrt(var + 1e-5)
    g0 = dm * inv * lnw_ref[0:1, :] + lnb_ref[0:1, :]
    g1 = dna * inv * lnw_ref[1:2, :] + lnb_ref[1:2, :]
    g2 = dns * inv * lnw_ref[2:3, :] + lnb_ref[2:3, :]
    h = (jnp.dot(g0, w1_ref[0], preferred_element_type=jnp.float32)
         + jnp.dot(g1, w1_ref[1], preferred_element_type=jnp.float32)
         + jnp.dot(g2, w1_ref[2], preferred_element_type=jnp.float32)
         + b1_ref[0:1, :])
    h = h * (1.0 / (1.0 + jnp.exp(-h)))
    cf = jnp.dot(h, w2_ref[...], preferred_element_type=jnp.float32)
    cf = cf + b2_ref[0:1, :]
    lane = lax.broadcasted_iota(jnp.int32, cf.shape, 1)
    cf_ref[...] = jnp.where(lane >= 16, cf * cf, cf)


def _stage_a(xp, lnw, lnb, w1, b1, w2, b2, npad):
    ntiles = npad // _TA
    return pl.pallas_call(
        _decompose_mlp_body,
        grid=(ntiles,),
        in_specs=[
            pl.BlockSpec((_TA, 9, 128), lambda i: (i, 0, 0)),
            pl.BlockSpec((3, 128), lambda i: (0, 0)),
            pl.BlockSpec((3, 128), lambda i: (0, 0)),
            pl.BlockSpec((3, 128, 128), lambda i: (0, 0, 0)),
            pl.BlockSpec((1, 128), lambda i: (0, 0)),
            pl.BlockSpec((128, 32), lambda i: (0, 0)),
            pl.BlockSpec((1, 32), lambda i: (0, 0)),
        ],
        out_specs=pl.BlockSpec((_TA, 32), lambda i: (i, 0)),
        out_shape=jax.ShapeDtypeStruct((npad, 32), jnp.float32),
    )(xp, lnw, lnb, w1, b1, w2, b2)


def _make_seg_sum(npad):
    ntiles = npad // _TS
    mesh = plsc.VectorSubcoreMesh(core_axis_name="c", subcore_axis_name="s")
    stripe_w = _ACC // 16

    @functools.partial(
        pl.kernel,
        mesh=mesh,
        out_type=jax.ShapeDtypeStruct((2 * _ACC,), jnp.float32),
        scratch_types=[
            pltpu.VMEM((_ACC,), jnp.float32),
            pltpu.VMEM((_TS * 32,), jnp.float32),
            pltpu.VMEM((_TS + 16,), jnp.int32),
            pltpu.VMEM((stripe_w,), jnp.float32),
            pltpu.VMEM((stripe_w,), jnp.float32),
            pltpu.VMEM_SHARED((16 * _ACC,), jnp.float32),
        ],
    )
    def seg_sum(cf_hbm, bat_hbm, part_hbm,
                acc_v, cf_v, bat_v, racc_v, rbuf_v, slots_sh):
        c = lax.axis_index("c")
        s = lax.axis_index("s")
        w = s * 2 + c
        zero16 = jnp.zeros((16,), jnp.float32)

        def _z(j, carry):
            acc_v[pl.ds(j * 16, 16)] = zero16
            return carry
        lax.fori_loop(0, _ACC // 16, _z, 0)

        cnt = ((ntiles - 1) - w) // _NW + 1

        def _tile(k, carry):
            tile = w + k * _NW
            pltpu.sync_copy(cf_hbm.at[pl.ds(tile * (_TS * 32), _TS * 32)],
                            cf_v.at[pl.ds(0, _TS * 32)])
            pltpu.sync_copy(bat_hbm.at[pl.ds(tile * _TS, _TS)],
                            bat_v.at[pl.ds(0, _TS)])

            def _atom(i, carry2):
                b = bat_v[pl.ds(i, 16)][0] * 32
                cvals = cf_v[pl.ds(i * 32, 16)]
                fvals = cf_v[pl.ds(i * 32 + 16, 16)]
                acc_v[pl.ds(b, 16)] = acc_v[pl.ds(b, 16)] + cvals
                acc_v[pl.ds(b + 16, 16)] = acc_v[pl.ds(b + 16, 16)] + fvals
                return carry2
            lax.fori_loop(0, _TS, _atom, 0)
            return carry
        lax.fori_loop(0, cnt, _tile, 0)

        pltpu.sync_copy(acc_v, slots_sh.at[pl.ds(s * _ACC, _ACC)])
        plsc.subcore_barrier()
        stripe = s * stripe_w
        pltpu.sync_copy(slots_sh.at[pl.ds(stripe, stripe_w)], racc_v)
        for k in range(1, 16):
            pltpu.sync_copy(slots_sh.at[pl.ds(k * _ACC + stripe, stripe_w)],
                            rbuf_v)

            def _add(j, carry):
                racc_v[pl.ds(j * 16, 16)] = (racc_v[pl.ds(j * 16, 16)]
                                             + rbuf_v[pl.ds(j * 16, 16)])
                return carry
            lax.fori_loop(0, stripe_w // 16, _add, 0)
        pltpu.sync_copy(racc_v, part_hbm.at[pl.ds(c * _ACC + stripe, stripe_w)])

    return seg_sum


def _make_combine(npad):
    ntiles = npad // _TS
    mesh = plsc.VectorSubcoreMesh(core_axis_name="c", subcore_axis_name="s")

    @functools.partial(
        pl.kernel,
        mesh=mesh,
        out_type=jax.ShapeDtypeStruct((npad * 16,), jnp.float32),
        scratch_types=[
            pltpu.VMEM((_ACC,), jnp.float32),
            pltpu.VMEM((_ACC,), jnp.float32),
            pltpu.VMEM((_TS * 32,), jnp.float32),
            pltpu.VMEM((_TS + 16,), jnp.int32),
            pltpu.VMEM((_TS + 16,), jnp.float32),
            pltpu.VMEM((_TS * 16,), jnp.float32),
        ],
    )
    def combine(cf_hbm, bat_hbm, q_hbm, part_hbm, out_hbm,
                sum_v, buf_v, cf_v, bat_v, q_v, out_v):
        c = lax.axis_index("c")
        s = lax.axis_index("s")
        w = s * 2 + c
        pltpu.sync_copy(part_hbm.at[pl.ds(0, _ACC)], sum_v)
        pltpu.sync_copy(part_hbm.at[pl.ds(_ACC, _ACC)], buf_v)

        def _add(j, carry):
            sum_v[pl.ds(j * 16, 16)] = (sum_v[pl.ds(j * 16, 16)]
                                        + buf_v[pl.ds(j * 16, 16)])
            return carry
        lax.fori_loop(0, _ACC // 16, _add, 0)

        cnt = ((ntiles - 1) - w) // _NW + 1

        def _tile(k, carry):
            tile = w + k * _NW
            base = tile * _TS
            pltpu.sync_copy(cf_hbm.at[pl.ds(base * 32, _TS * 32)], cf_v)
            pltpu.sync_copy(bat_hbm.at[pl.ds(base, _TS)],
                            bat_v.at[pl.ds(0, _TS)])
            pltpu.sync_copy(q_hbm.at[pl.ds(base, _TS)],
                            q_v.at[pl.ds(0, _TS)])

            def _atom(i, carry2):
                b = bat_v[pl.ds(i, 16)][0] * 32
                qu = sum_v[pl.ds(b, 16)]
                big_f = sum_v[pl.ds(b + 16, 16)]
                qs = q_v[pl.ds(i, 16)][0]
                ch = cf_v[pl.ds(i * 32, 16)]
                fu = cf_v[pl.ds(i * 32 + 16, 16)]
                out_v[pl.ds(i * 16, 16)] = (
                    ch + fu / (big_f + 1e-6) * (qs - qu))
                return carry2
            lax.fori_loop(0, _TS, _atom, 0)
            pltpu.sync_copy(out_v, out_hbm.at[pl.ds(base * 16, _TS * 16)])
            return carry
        lax.fori_loop(0, cnt, _tile, 0)

    return combine


def kernel(X, batch, Q, ln_w, ln_b, W1, b1, W2, b2):
    n, h = X.shape[0], X.shape[1]
    ntiles = -(-n // _TA)
    npad = ntiles * _TA
    xp = jnp.transpose(X.reshape(n, h, 9), (0, 2, 1))
    cf = _stage_a(xp, ln_w.reshape(3, h), ln_b.reshape(3, h),
                  W1.reshape(3, h, h), b1.reshape(1, h), W2,
                  b2.reshape(1, 32), npad)
    cf_flat = cf.reshape(-1)
    bat = jnp.pad(batch.astype(jnp.int32), (0, npad - n),
                  constant_values=_B)
    qpad = jnp.pad(Q, (0, npad - n))
    partials = _make_seg_sum(npad)(cf_flat, bat)
    out = _make_combine(npad)(cf_flat, bat, qpad, partials)
    return out.reshape(npad, 16)[:n]


# speedup vs baseline: 3.4866x; 2.7000x over previous
import functools

import jax
import jax.numpy as jnp
from jax import lax
from jax.experimental import pallas as pl
from jax.experimental.pallas import tpu as pltpu
from jax.experimental.pallas import tpu_sc as plsc

_B = 1000
_BPAD = 1024
_TA = 1024
_TS = 512
_NW = 32
_ACC = _BPAD * 32


def _decompose_mlp_body(x_ref, lnw_ref, lnb_ref, w1_ref, b1_ref, w2_ref,
                        b2_ref, cf_ref):
    xs = [x_ref[e] for e in range(9)]
    m = (xs[0] + xs[4] + xs[8]) * (1.0 / 3.0)
    d1 = xs[1] - xs[3]
    d2 = xs[2] - xs[6]
    d5 = xs[5] - xs[7]
    na = 0.5 * (d1 * d1 + d2 * d2 + d5 * d5)
    s1 = xs[1] + xs[3]
    s2 = xs[2] + xs[6]
    s5 = xs[5] + xs[7]
    t0 = xs[0] - m
    t4 = xs[4] - m
    t8 = xs[8] - m
    ns = t0 * t0 + t4 * t4 + t8 * t8 + 0.5 * (s1 * s1 + s2 * s2 + s5 * s5)
    ssum = (jnp.sum(m, axis=1, keepdims=True)
            + jnp.sum(na, axis=1, keepdims=True)
            + jnp.sum(ns, axis=1, keepdims=True))
    mu = ssum * (1.0 / 384.0)
    dm = m - mu
    dna = na - mu
    dns = ns - mu
    var = (jnp.sum(dm * dm, axis=1, keepdims=True)
           + jnp.sum(dna * dna, axis=1, keepdims=True)
           + jnp.sum(dns * dns, axis=1, keepdims=True)) * (1.0 / 384.0)
    inv = lax.rsqrt(var + 1e-5)
    g0 = dm * inv * lnw_ref[0:1, :] + lnb_ref[0:1, :]
    g1 = dna * inv * lnw_ref[1:2, :] + lnb_ref[1:2, :]
    g2 = dns * inv * lnw_ref[2:3, :] + lnb_ref[2:3, :]
    h = (jnp.dot(g0, w1_ref[0], preferred_element_type=jnp.float32)
         + jnp.dot(g1, w1_ref[1], preferred_element_type=jnp.float32)
         + jnp.dot(g2, w1_ref[2], preferred_element_type=jnp.float32)
         + b1_ref[0:1, :])
    h = h * (1.0 / (1.0 + jnp.exp(-h)))
    cf = jnp.dot(h, w2_ref[...], preferred_element_type=jnp.float32)
    cf = cf + b2_ref[0:1, :]
    lane = lax.broadcasted_iota(jnp.int32, cf.shape, 1)
    cf_ref[...] = jnp.where(lane >= 16, cf * cf, cf)


def _stage_a(xp, lnw, lnb, w1, b1, w2, b2, npad):
    ntiles = npad // _TA
    return pl.pallas_call(
        _decompose_mlp_body,
        grid=(ntiles,),
        in_specs=[
            pl.BlockSpec((9, _TA, 128), lambda i: (0, i, 0)),
            pl.BlockSpec((3, 128), lambda i: (0, 0)),
            pl.BlockSpec((3, 128), lambda i: (0, 0)),
            pl.BlockSpec((3, 128, 128), lambda i: (0, 0, 0)),
            pl.BlockSpec((1, 128), lambda i: (0, 0)),
            pl.BlockSpec((128, 32), lambda i: (0, 0)),
            pl.BlockSpec((1, 32), lambda i: (0, 0)),
        ],
        out_specs=pl.BlockSpec((_TA, 32), lambda i: (i, 0)),
        out_shape=jax.ShapeDtypeStruct((npad, 32), jnp.float32),
    )(xp, lnw, lnb, w1, b1, w2, b2)


def _make_seg_sum(npad):
    ntiles = npad // _TS
    mesh = plsc.VectorSubcoreMesh(core_axis_name="c", subcore_axis_name="s")
    stripe_w = _ACC // 16

    @functools.partial(
        pl.kernel,
        mesh=mesh,
        out_type=jax.ShapeDtypeStruct((2 * _ACC,), jnp.float32),
        scratch_types=[
            pltpu.VMEM((_ACC,), jnp.float32),
            pltpu.VMEM((_TS * 32,), jnp.float32),
            pltpu.VMEM((_TS + 16,), jnp.int32),
            pltpu.VMEM((stripe_w,), jnp.float32),
            pltpu.VMEM((stripe_w,), jnp.float32),
            pltpu.VMEM_SHARED((16 * _ACC,), jnp.float32),
        ],
    )
    def seg_sum(cf_hbm, bat_hbm, part_hbm,
                acc_v, cf_v, bat_v, racc_v, rbuf_v, slots_sh):
        c = lax.axis_index("c")
        s = lax.axis_index("s")
        w = s * 2 + c
        zero16 = jnp.zeros((16,), jnp.float32)

        def _z(j, carry):
            acc_v[pl.ds(j * 16, 16)] = zero16
            return carry
        lax.fori_loop(0, _ACC // 16, _z, 0)

        cnt = ((ntiles - 1) - w) // _NW + 1

        def _tile(k, carry):
            tile = w + k * _NW
            pltpu.sync_copy(cf_hbm.at[pl.ds(tile * (_TS * 32), _TS * 32)],
                            cf_v.at[pl.ds(0, _TS * 32)])
            pltpu.sync_copy(bat_hbm.at[pl.ds(tile * _TS, _TS)],
                            bat_v.at[pl.ds(0, _TS)])

            def _atom(i, carry2):
                b = bat_v[pl.ds(i, 16)][0] * 32
                cvals = cf_v[pl.ds(i * 32, 16)]
                fvals = cf_v[pl.ds(i * 32 + 16, 16)]
                acc_v[pl.ds(b, 16)] = acc_v[pl.ds(b, 16)] + cvals
                acc_v[pl.ds(b + 16, 16)] = acc_v[pl.ds(b + 16, 16)] + fvals
                return carry2
            lax.fori_loop(0, _TS, _atom, 0, unroll=8)
            return carry
        lax.fori_loop(0, cnt, _tile, 0)

        pltpu.sync_copy(acc_v, slots_sh.at[pl.ds(s * _ACC, _ACC)])
        plsc.subcore_barrier()
        stripe = s * stripe_w
        pltpu.sync_copy(slots_sh.at[pl.ds(stripe, stripe_w)], racc_v)
        for k in range(1, 16):
            pltpu.sync_copy(slots_sh.at[pl.ds(k * _ACC + stripe, stripe_w)],
                            rbuf_v)

            def _add(j, carry):
                racc_v[pl.ds(j * 16, 16)] = (racc_v[pl.ds(j * 16, 16)]
                                             + rbuf_v[pl.ds(j * 16, 16)])
                return carry
            lax.fori_loop(0, stripe_w // 16, _add, 0)
        pltpu.sync_copy(racc_v, part_hbm.at[pl.ds(c * _ACC + stripe, stripe_w)])

    return seg_sum


def _make_combine(npad):
    ntiles = npad // _TS
    mesh = plsc.VectorSubcoreMesh(core_axis_name="c", subcore_axis_name="s")

    @functools.partial(
        pl.kernel,
        mesh=mesh,
        out_type=jax.ShapeDtypeStruct((npad * 16,), jnp.float32),
        scratch_types=[
            pltpu.VMEM((_ACC,), jnp.float32),
            pltpu.VMEM((_ACC,), jnp.float32),
            pltpu.VMEM((_TS * 32,), jnp.float32),
            pltpu.VMEM((_TS + 16,), jnp.int32),
            pltpu.VMEM((_TS + 16,), jnp.float32),
            pltpu.VMEM((_TS * 16,), jnp.float32),
        ],
    )
    def combine(cf_hbm, bat_hbm, q_hbm, part_hbm, out_hbm,
                sum_v, buf_v, cf_v, bat_v, q_v, out_v):
        c = lax.axis_index("c")
        s = lax.axis_index("s")
        w = s * 2 + c
        pltpu.sync_copy(part_hbm.at[pl.ds(0, _ACC)], sum_v)
        pltpu.sync_copy(part_hbm.at[pl.ds(_ACC, _ACC)], buf_v)

        def _add(j, carry):
            sum_v[pl.ds(j * 16, 16)] = (sum_v[pl.ds(j * 16, 16)]
                                        + buf_v[pl.ds(j * 16, 16)])
            return carry
        lax.fori_loop(0, _ACC // 16, _add, 0)

        cnt = ((ntiles - 1) - w) // _NW + 1

        def _tile(k, carry):
            tile = w + k * _NW
            base = tile * _TS
            pltpu.sync_copy(cf_hbm.at[pl.ds(base * 32, _TS * 32)], cf_v)
            pltpu.sync_copy(bat_hbm.at[pl.ds(base, _TS)],
                            bat_v.at[pl.ds(0, _TS)])
            pltpu.sync_copy(q_hbm.at[pl.ds(base, _TS)],
                            q_v.at[pl.ds(0, _TS)])

            def _atom(i, carry2):
                b = bat_v[pl.ds(i, 16)][0] * 32
                qu = sum_v[pl.ds(b, 16)]
                big_f = sum_v[pl.ds(b + 16, 16)]
                qs = q_v[pl.ds(i, 16)][0]
                ch = cf_v[pl.ds(i * 32, 16)]
                fu = cf_v[pl.ds(i * 32 + 16, 16)]
                out_v[pl.ds(i * 16, 16)] = (
                    ch + fu / (big_f + 1e-6) * (qs - qu))
                return carry2
            lax.fori_loop(0, _TS, _atom, 0, unroll=8)
            pltpu.sync_copy(out_v, out_hbm.at[pl.ds(base * 16, _TS * 16)])
            return carry
        lax.fori_loop(0, cnt, _tile, 0)

    return combine


def kernel(X, batch, Q, ln_w, ln_b, W1, b1, W2, b2):
    n, h = X.shape[0], X.shape[1]
    ntiles = -(-n // _TA)
    npad = ntiles * _TA
    xp = jnp.transpose(X.reshape(n, h, 9), (2, 0, 1))
    cf = _stage_a(xp, ln_w.reshape(3, h), ln_b.reshape(3, h),
                  W1.reshape(3, h, h), b1.reshape(1, h), W2,
                  b2.reshape(1, 32), npad)
    cf_flat = cf.reshape(-1)
    bat = jnp.pad(batch.astype(jnp.int32), (0, npad - n),
                  constant_values=_B)
    qpad = jnp.pad(Q, (0, npad - n))
    partials = _make_seg_sum(npad)(cf_flat, bat)
    out = _make_combine(npad)(cf_flat, bat, qpad, partials)
    return out.reshape(npad, 16)[:n]
